# baseline (speedup 1.0000x reference)
import functools

import jax
import jax.numpy as jnp
from jax import lax
from jax.experimental import pallas as pl
from jax.experimental.pallas import tpu as pltpu
from jax.experimental.pallas import tpu_sc as plsc

N = 10000
NUM_MESHES = 16
NC = 2
NS = 16
B = 96
N_PAD = 10112
RPT = N_PAD // NS
DUMMY = 10008
EPS = 1e-5
RING = 3


G = 24


@functools.lru_cache(maxsize=None)
def _make_sc_scatter(C_PAD):
    mesh = plsc.VectorSubcoreMesh(core_axis_name="c", subcore_axis_name="s")

    NG = C_PAD // G

    @functools.partial(
        pl.kernel,
        out_type=jax.ShapeDtypeStruct((NC * N_PAD, 128), jnp.float32),
        mesh=mesh,
        scratch_types=[
            pltpu.VMEM((2, G, B), jnp.int32),
            pltpu.VMEM((2, G, B), jnp.int32),
            pltpu.VMEM((RING, B, 128), jnp.float32),
            pltpu.VMEM_SHARED((N_PAD, 128), jnp.float32),
            pltpu.SemaphoreType.DMA,
            pltpu.SemaphoreType.DMA,
            pltpu.SemaphoreType.DMA,
        ],
    )
    def sc_scatter(table_hbm, gidx_hbm, sidx_hbm, zero_hbm, out_hbm,
                   gsl, ssl, rows, acc_sh, gsem, ssem, s2sem):
        cid = lax.axis_index("c")
        sid = lax.axis_index("s")
        widx = cid * NS + sid
        r0 = sid * RPT
        grow0 = widx * C_PAD
        srow0 = sid * C_PAD
        pltpu.sync_copy(zero_hbm, acc_sh.at[pl.ds(r0, RPT)])
        plsc.subcore_barrier()
        pltpu.sync_copy(gidx_hbm.at[pl.ds(grow0, G)], gsl.at[0])
        pltpu.sync_copy(sidx_hbm.at[pl.ds(srow0, G)], ssl.at[0])
        if NG > 1:
            pltpu.async_copy(gidx_hbm.at[pl.ds(grow0 + G, G)], gsl.at[1],
                             ssem)
            pltpu.async_copy(sidx_hbm.at[pl.ds(srow0 + G, G)], ssl.at[1],
                             ssem)
        pltpu.async_copy(table_hbm.at[gsl.at[0, 0]], rows.at[0], gsem)
        pltpu.async_copy(table_hbm.at[gsl.at[0, 1]], rows.at[1], gsem)

        def group2(gi2, carry):
            for b in range(2):
                gi = gi2 + b
                for t in range(G):
                    q = t % RING
                    pltpu.make_async_copy(
                        table_hbm.at[pl.ds(0, B)], rows.at[q], gsem).wait()
                    if t >= 1:
                        pltpu.make_async_copy(
                            table_hbm.at[pl.ds(0, B)],
                            rows.at[(t - 1) % RING], s2sem).wait()
                    if t < G - 2:
                        pltpu.async_copy(table_hbm.at[gsl.at[b, t + 2]],
                                         rows.at[(t + 2) % RING], gsem)
                    elif t == G - 2:
                        @pl.when(gi + 1 < NG)
                        def _():
                            pltpu.make_async_copy(
                                gidx_hbm.at[pl.ds(0, G)], gsl.at[1 - b],
                                ssem).wait()
                            pltpu.make_async_copy(
                                sidx_hbm.at[pl.ds(0, G)], ssl.at[1 - b],
                                ssem).wait()
                            pltpu.async_copy(
                                table_hbm.at[gsl.at[1 - b, 0]],
                                rows.at[(t + 2) % RING], gsem)
                    else:
                        @pl.when(gi + 1 < NG)
                        def _():
                            pltpu.async_copy(
                                table_hbm.at[gsl.at[1 - b, 1]],
                                rows.at[(t + 2) % RING], gsem)
                    if t < G - 1:
                        pltpu.async_copy(rows.at[q],
                                         acc_sh.at[ssl.at[b, t]], s2sem,
                                         add=True)
                    else:
                        pltpu.sync_copy(rows.at[q],
                                        acc_sh.at[ssl.at[b, t]], add=True)

                        @pl.when(gi + 2 < NG)
                        def _():
                            pltpu.async_copy(
                                gidx_hbm.at[
                                    pl.ds(grow0 + (gi + 2) * G, G)],
                                gsl.at[b], ssem)
                            pltpu.async_copy(
                                sidx_hbm.at[
                                    pl.ds(srow0 + (gi + 2) * G, G)],
                                ssl.at[b], ssem)
            return carry

        lax.fori_loop(0, NG // 2, lambda i, c: group2(i * 2, c), 0)
        plsc.subcore_barrier()
        pltpu.sync_copy(acc_sh.at[pl.ds(r0, RPT)],
                        out_hbm.at[pl.ds(cid * N_PAD + r0, RPT)])

    return sc_scatter


def _mm2_body(x_ref, w0_ref, b0_ref, w1_ref, b1_ref, o0_ref, o1_ref):
    xv = x_ref[...]
    o0_ref[...] = (
        jnp.dot(xv, w0_ref[...], preferred_element_type=jnp.float32)
        + b0_ref[...]
    )
    vw1 = (
        jnp.dot(xv, w1_ref[...], preferred_element_type=jnp.float32)
        + b1_ref[...]
    )
    o1_ref[0] = vw1[:, :128]
    o1_ref[1] = vw1[:, 128:]


def _mm2(x, w0t, b0, w1t, b1):
    m = x.shape[0]
    return pl.pallas_call(
        _mm2_body,
        out_shape=[
            jax.ShapeDtypeStruct((m, 256), jnp.float32),
            jax.ShapeDtypeStruct((2, m, 128), jnp.float32),
        ],
    )(x, w0t, b0[None, :], w1t, b1[None, :])


def _bnrelu_body(vw0_ref, agg_ref, g_ref, be_ref, o_ref):
    t = vw0_ref[...] + jnp.concatenate(
        [agg_ref[0, :N, :], agg_ref[1, :N, :]], axis=1)
    mu = jnp.mean(t, axis=0, keepdims=True)
    xc = t - mu
    var = jnp.mean(xc * xc, axis=0, keepdims=True)
    scale = g_ref[...] / jnp.sqrt(var + EPS)
    o_ref[...] = jnp.maximum(xc * scale + be_ref[...], 0.0)


def _bnrelu(vw0, agg, g, be):
    return pl.pallas_call(
        _bnrelu_body,
        out_shape=jax.ShapeDtypeStruct((N, 256), jnp.float32),
    )(vw0, agg, g[None, :], be[None, :])


def _pool_heads_body(x_ref, vi_ref,
                     ws1_ref, bs1_ref, ws2_ref, bs2_ref,
                     wf1_ref, bf1_ref, wf2_ref, bf2_ref,
                     wa1_ref, ba1_ref, wa2_ref, ba2_ref,
                     os_ref, of_ref, oa_ref):
    vi = vi_ref[...]
    seg = lax.broadcasted_iota(jnp.int32, (NUM_MESHES, 1), 0)
    onehot = (vi == seg).astype(jnp.float32)
    counts = jnp.sum(onehot, axis=1, keepdims=True)
    sums = jnp.dot(onehot, x_ref[...], preferred_element_type=jnp.float32)
    pooled = sums / jnp.maximum(counts, 1.0)

    def head(w1_ref, b1_ref, w2_ref, b2_ref):
        h = jnp.maximum(
            jnp.dot(pooled, w1_ref[...], preferred_element_type=jnp.float32)
            + b1_ref[...], 0.0)
        return (
            jnp.dot(h, w2_ref[...], preferred_element_type=jnp.float32)
            + b2_ref[...]
        )

    os_ref[...] = head(ws1_ref, bs1_ref, ws2_ref, bs2_ref)
    of_ref[...] = head(wf1_ref, bf1_ref, wf2_ref, bf2_ref)
    oa_ref[...] = head(wa1_ref, ba1_ref, wa2_ref, ba2_ref)


def _pool_heads(x, vi, ws1, bs1, ws2, bs2, wf1, bf1, wf2, bf2,
                wa1, ba1, wa2, ba2):
    return pl.pallas_call(
        _pool_heads_body,
        out_shape=[
            jax.ShapeDtypeStruct((NUM_MESHES, ws2.shape[0]), jnp.float32),
            jax.ShapeDtypeStruct((NUM_MESHES, wf2.shape[0]), jnp.float32),
            jax.ShapeDtypeStruct((NUM_MESHES, wa2.shape[0]), jnp.float32),
        ],
    )(x, vi,
      ws1.T, bs1[None, :], ws2.T, bs2[None, :],
      wf1.T, bf1[None, :], wf2.T, bf2[None, :],
      wa1.T, ba1[None, :], wa2.T, ba2[None, :])


def kernel(verts, edges, verts_idx,
           w0_0, b0_0, w1_0, b1_0, g_0, be_0,
           w0_1, b0_1, w1_1, b1_1, g_1, be_1,
           ws1, bs1, ws2, bs2, wf1, bf1, wf2, bf2, wa1, ba1, wa2, ba2):
    e = edges.astype(jnp.int32)
    e2 = 2 * e.shape[0]
    per_tile = -(-e2 // NS)
    C = -(-per_tile // B)
    C_PAD = -(-C // (2 * G)) * (2 * G)
    total = NS * C_PAD * B
    pad = total - e2

    g_all = jnp.concatenate(
        [e[:, 1], e[:, 0],
         jnp.zeros((pad,), jnp.int32)]).reshape(NS * C_PAD, B)
    s_all = jnp.concatenate(
        [e[:, 0], e[:, 1],
         jnp.full((pad,), DUMMY, jnp.int32)]).reshape(NS * C_PAD, B)
    gidx = jnp.concatenate([g_all, g_all + N], axis=0)
    zero = jnp.zeros((RPT, 128), jnp.float32)

    sc_scatter = _make_sc_scatter(C_PAD)

    x = verts
    layers = [(w0_0, b0_0, w1_0, b1_0, g_0, be_0),
              (w0_1, b0_1, w1_1, b1_1, g_1, be_1)]
    for (w0, b0, w1, b1, g, be) in layers:
        vw0, vw1h = _mm2(x, w0.T, b0, w1.T, b1)
        agg = sc_scatter(vw1h.reshape(NC * N, 128), gidx, s_all, zero)
        x = _bnrelu(vw0, agg.reshape(NC, N_PAD, 128), g, be)

    vi = verts_idx.astype(jnp.int32).reshape(1, N)
    return _pool_heads(x, vi, ws1, bs1, ws2, bs2,
                       wf1, bf1, wf2, bf2, wa1, ba1, wa2, ba2)


# speedup vs baseline: 1.3346x; 1.3346x over previous
import functools

import jax
import jax.numpy as jnp
from jax import lax
from jax.experimental import pallas as pl
from jax.experimental.pallas import tpu as pltpu
from jax.experimental.pallas import tpu_sc as plsc

N = 10000
NUM_MESHES = 16
NC = 2
NS = 16
B = 64
N_PAD = 10112
RPT = N_PAD // NS
DUMMY = 10008
EPS = 1e-5
RING = 4
IN_FLIGHT = RING - 1


G = 32


@functools.lru_cache(maxsize=None)
def _make_sc_scatter(C_PAD):
    mesh = plsc.VectorSubcoreMesh(core_axis_name="c", subcore_axis_name="s")

    NG = C_PAD // G

    @functools.partial(
        pl.kernel,
        out_type=jax.ShapeDtypeStruct((NC * N_PAD, 128), jnp.float32),
        mesh=mesh,
        scratch_types=[
            pltpu.VMEM((2, G, B), jnp.int32),
            pltpu.VMEM((2, G, B), jnp.int32),
            pltpu.VMEM((RING, B, 128), jnp.float32),
            pltpu.VMEM_SHARED((N_PAD, 128), jnp.float32),
            pltpu.SemaphoreType.DMA,
            pltpu.SemaphoreType.DMA,
            pltpu.SemaphoreType.DMA,
        ],
    )
    def sc_scatter(table_hbm, gidx_hbm, sidx_hbm, zero_hbm, out_hbm,
                   gsl, ssl, rows, acc_sh, gsem, ssem, s2sem):
        cid = lax.axis_index("c")
        sid = lax.axis_index("s")
        widx = cid * NS + sid
        r0 = sid * RPT
        grow0 = widx * C_PAD
        srow0 = sid * C_PAD
        pltpu.sync_copy(zero_hbm, acc_sh.at[pl.ds(r0, RPT)])
        plsc.subcore_barrier()
        pltpu.sync_copy(gidx_hbm.at[pl.ds(grow0, G)], gsl.at[0])
        pltpu.sync_copy(sidx_hbm.at[pl.ds(srow0, G)], ssl.at[0])
        if NG > 1:
            pltpu.async_copy(gidx_hbm.at[pl.ds(grow0 + G, G)], gsl.at[1],
                             ssem)
            pltpu.async_copy(sidx_hbm.at[pl.ds(srow0 + G, G)], ssl.at[1],
                             ssem)
        for k in range(IN_FLIGHT):
            pltpu.async_copy(table_hbm.at[gsl.at[0, k]], rows.at[k], gsem)

        def group2(gi2, carry):
            for b in range(2):
                gi = gi2 + b
                for t in range(G):
                    q = t % RING
                    pltpu.make_async_copy(
                        table_hbm.at[pl.ds(0, B)], rows.at[q], gsem).wait()
                    if t >= 1:
                        pltpu.make_async_copy(
                            table_hbm.at[pl.ds(0, B)],
                            rows.at[(t - 1) % RING], s2sem).wait()
                    nt = t + IN_FLIGHT
                    if nt < G:
                        pltpu.async_copy(table_hbm.at[gsl.at[b, nt]],
                                         rows.at[nt % RING], gsem)
                    else:
                        @pl.when(gi + 1 < NG)
                        def _():
                            if nt == G:
                                pltpu.make_async_copy(
                                    gidx_hbm.at[pl.ds(0, G)],
                                    gsl.at[1 - b], ssem).wait()
                                pltpu.make_async_copy(
                                    sidx_hbm.at[pl.ds(0, G)],
                                    ssl.at[1 - b], ssem).wait()
                            pltpu.async_copy(
                                table_hbm.at[gsl.at[1 - b, nt - G]],
                                rows.at[nt % RING], gsem)
                    if t < G - 1:
                        pltpu.async_copy(rows.at[q],
                                         acc_sh.at[ssl.at[b, t]], s2sem,
                                         add=True)
                    else:
                        pltpu.sync_copy(rows.at[q],
                                        acc_sh.at[ssl.at[b, t]], add=True)

                        @pl.when(gi + 2 < NG)
                        def _():
                            pltpu.async_copy(
                                gidx_hbm.at[
                                    pl.ds(grow0 + (gi + 2) * G, G)],
                                gsl.at[b], ssem)
                            pltpu.async_copy(
                                sidx_hbm.at[
                                    pl.ds(srow0 + (gi + 2) * G, G)],
                                ssl.at[b], ssem)
            return carry

        lax.fori_loop(0, NG // 2, lambda i, c: group2(i * 2, c), 0)
        plsc.subcore_barrier()
        pltpu.sync_copy(acc_sh.at[pl.ds(r0, RPT)],
                        out_hbm.at[pl.ds(cid * N_PAD + r0, RPT)])

    return sc_scatter


def _mm2_body(x_ref, w0_ref, b0_ref, w1_ref, b1_ref, o0_ref, o1_ref):
    xv = x_ref[...]
    o0_ref[...] = (
        jnp.dot(xv, w0_ref[...], preferred_element_type=jnp.float32)
        + b0_ref[...]
    )
    vw1 = (
        jnp.dot(xv, w1_ref[...], preferred_element_type=jnp.float32)
        + b1_ref[...]
    )
    o1_ref[0] = vw1[:, :128]
    o1_ref[1] = vw1[:, 128:]


def _mm2(x, w0t, b0, w1t, b1):
    m = x.shape[0]
    return pl.pallas_call(
        _mm2_body,
        out_shape=[
            jax.ShapeDtypeStruct((m, 256), jnp.float32),
            jax.ShapeDtypeStruct((2, m, 128), jnp.float32),
        ],
    )(x, w0t, b0[None, :], w1t, b1[None, :])


def _bnrelu_body(vw0_ref, agg_ref, g_ref, be_ref, o_ref):
    t = vw0_ref[...] + jnp.concatenate(
        [agg_ref[0, :N, :], agg_ref[1, :N, :]], axis=1)
    mu = jnp.mean(t, axis=0, keepdims=True)
    xc = t - mu
    var = jnp.mean(xc * xc, axis=0, keepdims=True)
    scale = g_ref[...] / jnp.sqrt(var + EPS)
    o_ref[...] = jnp.maximum(xc * scale + be_ref[...], 0.0)


def _bnrelu(vw0, agg, g, be):
    return pl.pallas_call(
        _bnrelu_body,
        out_shape=jax.ShapeDtypeStruct((N, 256), jnp.float32),
    )(vw0, agg, g[None, :], be[None, :])


def _pool_heads_body(x_ref, vi_ref,
                     ws1_ref, bs1_ref, ws2_ref, bs2_ref,
                     wf1_ref, bf1_ref, wf2_ref, bf2_ref,
                     wa1_ref, ba1_ref, wa2_ref, ba2_ref,
                     os_ref, of_ref, oa_ref):
    vi = vi_ref[...]
    seg = lax.broadcasted_iota(jnp.int32, (NUM_MESHES, 1), 0)
    onehot = (vi == seg).astype(jnp.float32)
    counts = jnp.sum(onehot, axis=1, keepdims=True)
    sums = jnp.dot(onehot, x_ref[...], preferred_element_type=jnp.float32)
    pooled = sums / jnp.maximum(counts, 1.0)

    def head(w1_ref, b1_ref, w2_ref, b2_ref):
        h = jnp.maximum(
            jnp.dot(pooled, w1_ref[...], preferred_element_type=jnp.float32)
            + b1_ref[...], 0.0)
        return (
            jnp.dot(h, w2_ref[...], preferred_element_type=jnp.float32)
            + b2_ref[...]
        )

    os_ref[...] = head(ws1_ref, bs1_ref, ws2_ref, bs2_ref)
    of_ref[...] = head(wf1_ref, bf1_ref, wf2_ref, bf2_ref)
    oa_ref[...] = head(wa1_ref, ba1_ref, wa2_ref, ba2_ref)


def _pool_heads(x, vi, ws1, bs1, ws2, bs2, wf1, bf1, wf2, bf2,
                wa1, ba1, wa2, ba2):
    return pl.pallas_call(
        _pool_heads_body,
        out_shape=[
            jax.ShapeDtypeStruct((NUM_MESHES, ws2.shape[0]), jnp.float32),
            jax.ShapeDtypeStruct((NUM_MESHES, wf2.shape[0]), jnp.float32),
            jax.ShapeDtypeStruct((NUM_MESHES, wa2.shape[0]), jnp.float32),
        ],
    )(x, vi,
      ws1.T, bs1[None, :], ws2.T, bs2[None, :],
      wf1.T, bf1[None, :], wf2.T, bf2[None, :],
      wa1.T, ba1[None, :], wa2.T, ba2[None, :])


def kernel(verts, edges, verts_idx,
           w0_0, b0_0, w1_0, b1_0, g_0, be_0,
           w0_1, b0_1, w1_1, b1_1, g_1, be_1,
           ws1, bs1, ws2, bs2, wf1, bf1, wf2, bf2, wa1, ba1, wa2, ba2):
    e = edges.astype(jnp.int32)
    e2 = 2 * e.shape[0]
    per_tile = -(-e2 // NS)
    C = -(-per_tile // B)
    C_PAD = -(-C // (2 * G)) * (2 * G)
    total = NS * C_PAD * B
    pad = total - e2

    g_all = jnp.concatenate(
        [e[:, 1], e[:, 0],
         jnp.zeros((pad,), jnp.int32)]).reshape(NS * C_PAD, B)
    s_all = jnp.concatenate(
        [e[:, 0], e[:, 1],
         jnp.full((pad,), DUMMY, jnp.int32)]).reshape(NS * C_PAD, B)
    gidx = jnp.concatenate([g_all, g_all + N], axis=0)
    zero = jnp.zeros((RPT, 128), jnp.float32)

    sc_scatter = _make_sc_scatter(C_PAD)

    x = verts
    layers = [(w0_0, b0_0, w1_0, b1_0, g_0, be_0),
              (w0_1, b0_1, w1_1, b1_1, g_1, be_1)]
    for (w0, b0, w1, b1, g, be) in layers:
        vw0, vw1h = _mm2(x, w0.T, b0, w1.T, b1)
        agg = sc_scatter(vw1h.reshape(NC * N, 128), gidx, s_all, zero)
        x = _bnrelu(vw0, agg.reshape(NC, N_PAD, 128), g, be)

    vi = verts_idx.astype(jnp.int32).reshape(1, N)
    return _pool_heads(x, vi, ws1, bs1, ws2, bs2,
                       wf1, bf1, wf2, bf2, wa1, ba1, wa2, ba2)
